# baseline (speedup 1.0000x reference)
import functools

import jax
import jax.numpy as jnp
from jax import lax
from jax.experimental import pallas as pl
from jax.experimental.pallas import tpu as pltpu
from jax.experimental.pallas import tpu_sc as plsc

N_NODES = 10000
N_EDGES = 320000
D_FEAT = 128
DELTA_X = 0.1
SCALE = 1.0 / (DELTA_X * DELTA_X)

NS = 16
N_PAD = 10240
NPT = N_PAD // NS
NLAST = N_NODES - (NS - 1) * NPT
LANES = 16
ECHUNKS = N_EDGES // 128
CPW = ECHUNKS // NS
CH1 = CPW // 2
CREM = ECHUNKS - CPW * NS
UNROLL = 8


def _sc_body(x_hbm, ei_hbm, out_hbm,
             xcol_sh, xcol_v, gidx_v, gath_v, eiv, acc_v, tmpa_v, tmpb_v,
             racc_v, shared, sem, sem2, semb):
    sid = lax.axis_index("s")
    r0 = sid * NPT
    t0 = sid * CPW

    cpe1 = pltpu.async_copy(ei_hbm.at[pl.ds(t0, CH1)], eiv.at[pl.ds(0, CH1)], sem)
    cpe2 = pltpu.async_copy(ei_hbm.at[pl.ds(t0 + CH1, CPW - CH1)],
                            eiv.at[pl.ds(CH1, CPW - CH1)], semb)

    lanes = lax.iota(jnp.int32, LANES)
    for k in range(NPT // 128):
        for c in range(128 // LANES):
            j0 = k * 128 + c * LANES
            n = jnp.minimum(lanes + (r0 + j0), N_NODES - 1)
            gidx_v[k, pl.ds(c * LANES, LANES)] = n * D_FEAT
    gcps = [pltpu.async_copy(x_hbm.at[gidx_v.at[k]],
                             gath_v.at[pl.ds(k * 128, 128)], sem2)
            for k in range(NPT // 128)]

    @pl.when(sid < CREM)
    def _():
        pltpu.async_copy(ei_hbm.at[pl.ds(NS * CPW + sid, 1)],
                         eiv.at[pl.ds(CPW, 1)], sem).wait()

    zeros = jnp.zeros((LANES,), jnp.float32)

    def zero_body(i):
        acc_v[pl.ds(i * LANES, LANES)] = zeros
    plsc.parallel_loop(0, N_PAD // LANES, unroll=4)(zero_body)

    for cp in gcps:
        cp.wait()
    pltpu.sync_copy(gath_v, xcol_sh.at[pl.ds(r0, NPT)])
    plsc.subcore_barrier()
    bcast = pltpu.async_copy(xcol_sh, xcol_v, sem2)
    cpe1.wait()
    bcast.wait()

    def edge_chunk(t):
        for c in range(128 // LANES):
            s = eiv[t, 0, pl.ds(c * LANES, LANES)]
            d = eiv[t, 1, pl.ds(c * LANES, LANES)]
            vals = plsc.load_gather(xcol_v, [s])
            plsc.addupdate_scatter(acc_v, [d], vals)
    plsc.parallel_loop(0, CH1, unroll=UNROLL)(edge_chunk)
    cpe2.wait()
    plsc.parallel_loop(CH1, CPW, unroll=UNROLL)(edge_chunk)

    @pl.when(sid < CREM)
    def _():
        edge_chunk(CPW)

    pltpu.sync_copy(acc_v, shared.at[sid])
    plsc.subcore_barrier()

    pltpu.make_async_copy(shared.at[1, pl.ds(r0, NPT)], tmpa_v, sem2).start()
    pltpu.sync_copy(shared.at[0, pl.ds(r0, NPT)], racc_v)

    def racc_add(src_v):
        def add_body(j):
            jo = j * LANES
            racc_v[pl.ds(jo, LANES)] = racc_v[pl.ds(jo, LANES)] + src_v[pl.ds(jo, LANES)]
        plsc.parallel_loop(0, NPT // LANES, unroll=4)(add_body)

    def red_pair(u, _):
        t = 1 + 2 * u
        pltpu.make_async_copy(shared.at[t + 1, pl.ds(r0, NPT)], tmpb_v, semb).start()
        pltpu.make_async_copy(shared.at[t, pl.ds(r0, NPT)], tmpa_v, sem2).wait()
        racc_add(tmpa_v)
        pltpu.make_async_copy(shared.at[t + 2, pl.ds(r0, NPT)], tmpa_v, sem2).start()
        pltpu.make_async_copy(shared.at[t + 1, pl.ds(r0, NPT)], tmpb_v, semb).wait()
        racc_add(tmpb_v)
        return 0
    lax.fori_loop(0, (NS - 2) // 2, red_pair, 0)

    pltpu.make_async_copy(shared.at[NS - 1, pl.ds(r0, NPT)], tmpa_v, sem2).wait()
    racc_add(tmpa_v)

    def ep_body(j):
        jo = j * LANES
        racc_v[pl.ds(jo, LANES)] = (racc_v[pl.ds(jo, LANES)]
                                    - 2.0 * gath_v[pl.ds(jo, LANES)]) * SCALE
    plsc.parallel_loop(0, NPT // LANES, unroll=4)(ep_body)

    @pl.when(sid < NS - 1)
    def _():
        pltpu.sync_copy(racc_v, out_hbm.at[pl.ds(r0, NPT)])

    @pl.when(sid == NS - 1)
    def _():
        pltpu.sync_copy(racc_v.at[pl.ds(0, NLAST)],
                        out_hbm.at[pl.ds((NS - 1) * NPT, NLAST)])


@jax.jit
def _sc_scatter(x_flat, ei_blk):
    mesh = plsc.VectorSubcoreMesh(core_axis_name="c", subcore_axis_name="s",
                                  num_cores=1)
    return pl.kernel(
        _sc_body,
        out_type=jax.ShapeDtypeStruct((N_NODES,), jnp.float32),
        mesh=mesh,
        compiler_params=pltpu.CompilerParams(needs_layout_passes=False),
        scratch_types=[
            pltpu.VMEM_SHARED((N_PAD,), jnp.float32),
            pltpu.VMEM((N_PAD,), jnp.float32),
            pltpu.VMEM((NPT // 128, 128), jnp.int32),
            pltpu.VMEM((NPT,), jnp.float32),
            pltpu.VMEM((CPW + 1, 2, 128), jnp.int32),
            pltpu.VMEM((N_PAD,), jnp.float32),
            pltpu.VMEM((NPT,), jnp.float32),
            pltpu.VMEM((NPT,), jnp.float32),
            pltpu.VMEM((NPT,), jnp.float32),
            pltpu.VMEM_SHARED((NS, N_PAD), jnp.float32),
            pltpu.SemaphoreType.DMA,
            pltpu.SemaphoreType.DMA,
            pltpu.SemaphoreType.DMA,
        ],
    )(x_flat, ei_blk)


def kernel(x, edge_index, edge_attr):
    x_flat = x.reshape(-1)
    ei_blk = jnp.transpose(edge_index.reshape(2, ECHUNKS, 128), (1, 0, 2))
    return _sc_scatter(x_flat, ei_blk)


# speedup vs baseline: 1.0735x; 1.0735x over previous
import functools

import jax
import jax.numpy as jnp
from jax import lax
from jax.experimental import pallas as pl
from jax.experimental.pallas import tpu as pltpu
from jax.experimental.pallas import tpu_sc as plsc

N_NODES = 10000
N_EDGES = 320000
D_FEAT = 128
DELTA_X = 0.1
SCALE = 1.0 / (DELTA_X * DELTA_X)

NS = 16
N_PAD = 10240
NPT = N_PAD // NS
NLAST = N_NODES - (NS - 1) * NPT
LANES = 16
ECHUNKS = N_EDGES // 128
CPW = ECHUNKS // NS
CH1 = CPW // 2
CREM = ECHUNKS - CPW * NS
UNROLL = 2


def _sc_body(x_hbm, ei_hbm, out_hbm,
             xcol_sh, xcol_v, gidx_v, gath_v, eiv, acc_v, tmpa_v, tmpb_v,
             racc_v, shared, sem, sem2, semb):
    sid = lax.axis_index("s")
    r0 = sid * NPT
    t0 = sid * CPW

    cpe1 = pltpu.async_copy(ei_hbm.at[pl.ds(t0, CH1)], eiv.at[pl.ds(0, CH1)], sem)
    cpe2 = pltpu.async_copy(ei_hbm.at[pl.ds(t0 + CH1, CPW - CH1)],
                            eiv.at[pl.ds(CH1, CPW - CH1)], semb)

    lanes = lax.iota(jnp.int32, LANES)
    for k in range(NPT // 128):
        for c in range(128 // LANES):
            j0 = k * 128 + c * LANES
            n = jnp.minimum(lanes + (r0 + j0), N_NODES - 1)
            gidx_v[k, pl.ds(c * LANES, LANES)] = n * D_FEAT
    gcps = [pltpu.async_copy(x_hbm.at[gidx_v.at[k]],
                             gath_v.at[pl.ds(k * 128, 128)], sem2)
            for k in range(NPT // 128)]

    @pl.when(sid < CREM)
    def _():
        pltpu.async_copy(ei_hbm.at[pl.ds(NS * CPW + sid, 1)],
                         eiv.at[pl.ds(CPW, 1)], sem).wait()

    zeros = jnp.zeros((LANES,), jnp.float32)

    def zero_body(i):
        acc_v[pl.ds(i * LANES, LANES)] = zeros
    plsc.parallel_loop(0, N_PAD // LANES, unroll=4)(zero_body)

    for cp in gcps:
        cp.wait()
    pltpu.sync_copy(gath_v, xcol_sh.at[pl.ds(r0, NPT)])
    plsc.subcore_barrier()
    bcast = pltpu.async_copy(xcol_sh, xcol_v, sem2)
    cpe1.wait()
    bcast.wait()

    def edge_chunk(t):
        for c in range(128 // LANES):
            s = eiv[t, 0, pl.ds(c * LANES, LANES)]
            d = eiv[t, 1, pl.ds(c * LANES, LANES)]
            vals = plsc.load_gather(xcol_v, [s])
            plsc.addupdate_scatter(acc_v, [d], vals)
    plsc.parallel_loop(0, CH1, unroll=UNROLL)(edge_chunk)
    cpe2.wait()
    plsc.parallel_loop(CH1, CPW, unroll=UNROLL)(edge_chunk)

    @pl.when(sid < CREM)
    def _():
        edge_chunk(CPW)

    pltpu.sync_copy(acc_v, shared.at[sid])
    plsc.subcore_barrier()

    pltpu.make_async_copy(shared.at[1, pl.ds(r0, NPT)], tmpa_v, sem2).start()
    pltpu.sync_copy(shared.at[0, pl.ds(r0, NPT)], racc_v)

    def racc_add(src_v):
        def add_body(j):
            jo = j * LANES
            racc_v[pl.ds(jo, LANES)] = racc_v[pl.ds(jo, LANES)] + src_v[pl.ds(jo, LANES)]
        plsc.parallel_loop(0, NPT // LANES, unroll=4)(add_body)

    def red_pair(u, _):
        t = 1 + 2 * u
        pltpu.make_async_copy(shared.at[t + 1, pl.ds(r0, NPT)], tmpb_v, semb).start()
        pltpu.make_async_copy(shared.at[t, pl.ds(r0, NPT)], tmpa_v, sem2).wait()
        racc_add(tmpa_v)
        pltpu.make_async_copy(shared.at[t + 2, pl.ds(r0, NPT)], tmpa_v, sem2).start()
        pltpu.make_async_copy(shared.at[t + 1, pl.ds(r0, NPT)], tmpb_v, semb).wait()
        racc_add(tmpb_v)
        return 0
    lax.fori_loop(0, (NS - 2) // 2, red_pair, 0)

    pltpu.make_async_copy(shared.at[NS - 1, pl.ds(r0, NPT)], tmpa_v, sem2).wait()
    racc_add(tmpa_v)

    def ep_body(j):
        jo = j * LANES
        racc_v[pl.ds(jo, LANES)] = (racc_v[pl.ds(jo, LANES)]
                                    - 2.0 * gath_v[pl.ds(jo, LANES)]) * SCALE
    plsc.parallel_loop(0, NPT // LANES, unroll=4)(ep_body)

    @pl.when(sid < NS - 1)
    def _():
        pltpu.sync_copy(racc_v, out_hbm.at[pl.ds(r0, NPT)])

    @pl.when(sid == NS - 1)
    def _():
        pltpu.sync_copy(racc_v.at[pl.ds(0, NLAST)],
                        out_hbm.at[pl.ds((NS - 1) * NPT, NLAST)])


@jax.jit
def _sc_scatter(x_flat, ei_blk):
    mesh = plsc.VectorSubcoreMesh(core_axis_name="c", subcore_axis_name="s",
                                  num_cores=1)
    return pl.kernel(
        _sc_body,
        out_type=jax.ShapeDtypeStruct((N_NODES,), jnp.float32),
        mesh=mesh,
        compiler_params=pltpu.CompilerParams(needs_layout_passes=False),
        scratch_types=[
            pltpu.VMEM_SHARED((N_PAD,), jnp.float32),
            pltpu.VMEM((N_PAD,), jnp.float32),
            pltpu.VMEM((NPT // 128, 128), jnp.int32),
            pltpu.VMEM((NPT,), jnp.float32),
            pltpu.VMEM((CPW + 1, 2, 128), jnp.int32),
            pltpu.VMEM((N_PAD,), jnp.float32),
            pltpu.VMEM((NPT,), jnp.float32),
            pltpu.VMEM((NPT,), jnp.float32),
            pltpu.VMEM((NPT,), jnp.float32),
            pltpu.VMEM_SHARED((NS, N_PAD), jnp.float32),
            pltpu.SemaphoreType.DMA,
            pltpu.SemaphoreType.DMA,
            pltpu.SemaphoreType.DMA,
        ],
    )(x_flat, ei_blk)


def kernel(x, edge_index, edge_attr):
    x_flat = x.reshape(-1)
    ei_blk = jnp.transpose(edge_index.reshape(2, ECHUNKS, 128), (1, 0, 2))
    return _sc_scatter(x_flat, ei_blk)


# speedup vs baseline: 1.0850x; 1.0107x over previous
import functools

import jax
import jax.numpy as jnp
from jax import lax
from jax.experimental import pallas as pl
from jax.experimental.pallas import tpu as pltpu
from jax.experimental.pallas import tpu_sc as plsc

N_NODES = 10000
N_EDGES = 320000
D_FEAT = 128
DELTA_X = 0.1
SCALE = 1.0 / (DELTA_X * DELTA_X)

NS = 16
N_PAD = 10240
NPT = N_PAD // NS
NLAST = N_NODES - (NS - 1) * NPT
LANES = 16
ECHUNKS = N_EDGES // 128
CPW = ECHUNKS // NS
CH1 = CPW // 2
CREM = ECHUNKS - CPW * NS
UNROLL = 1


def _sc_body(x_hbm, ei_hbm, out_hbm,
             xcol_sh, xcol_v, gidx_v, gath_v, eiv, acc_v, tmpa_v, tmpb_v,
             racc_v, shared, sem, sem2, semb):
    sid = lax.axis_index("s")
    r0 = sid * NPT
    t0 = sid * CPW

    cpe1 = pltpu.async_copy(ei_hbm.at[pl.ds(t0, CH1)], eiv.at[pl.ds(0, CH1)], sem)
    cpe2 = pltpu.async_copy(ei_hbm.at[pl.ds(t0 + CH1, CPW - CH1)],
                            eiv.at[pl.ds(CH1, CPW - CH1)], semb)

    lanes = lax.iota(jnp.int32, LANES)
    for k in range(NPT // 128):
        for c in range(128 // LANES):
            j0 = k * 128 + c * LANES
            n = jnp.minimum(lanes + (r0 + j0), N_NODES - 1)
            gidx_v[k, pl.ds(c * LANES, LANES)] = n * D_FEAT
    gcps = [pltpu.async_copy(x_hbm.at[gidx_v.at[k]],
                             gath_v.at[pl.ds(k * 128, 128)], sem2)
            for k in range(NPT // 128)]

    @pl.when(sid < CREM)
    def _():
        pltpu.async_copy(ei_hbm.at[pl.ds(NS * CPW + sid, 1)],
                         eiv.at[pl.ds(CPW, 1)], sem).wait()

    zeros = jnp.zeros((LANES,), jnp.float32)

    def zero_body(i):
        acc_v[pl.ds(i * LANES, LANES)] = zeros
    plsc.parallel_loop(0, N_PAD // LANES, unroll=4)(zero_body)

    for cp in gcps:
        cp.wait()
    pltpu.sync_copy(gath_v, xcol_sh.at[pl.ds(r0, NPT)])
    plsc.subcore_barrier()
    bcast = pltpu.async_copy(xcol_sh, xcol_v, sem2)
    cpe1.wait()
    bcast.wait()

    def edge_chunk(t):
        for c in range(128 // LANES):
            s = eiv[t, 0, pl.ds(c * LANES, LANES)]
            d = eiv[t, 1, pl.ds(c * LANES, LANES)]
            vals = plsc.load_gather(xcol_v, [s])
            plsc.addupdate_scatter(acc_v, [d], vals)
    plsc.parallel_loop(0, CH1, unroll=UNROLL)(edge_chunk)
    cpe2.wait()
    plsc.parallel_loop(CH1, CPW, unroll=UNROLL)(edge_chunk)

    @pl.when(sid < CREM)
    def _():
        edge_chunk(CPW)

    pltpu.sync_copy(acc_v, shared.at[sid])
    plsc.subcore_barrier()

    pltpu.make_async_copy(shared.at[1, pl.ds(r0, NPT)], tmpa_v, sem2).start()
    pltpu.sync_copy(shared.at[0, pl.ds(r0, NPT)], racc_v)

    def racc_add(src_v):
        def add_body(j):
            jo = j * LANES
            racc_v[pl.ds(jo, LANES)] = racc_v[pl.ds(jo, LANES)] + src_v[pl.ds(jo, LANES)]
        plsc.parallel_loop(0, NPT // LANES, unroll=4)(add_body)

    def red_pair(u, _):
        t = 1 + 2 * u
        pltpu.make_async_copy(shared.at[t + 1, pl.ds(r0, NPT)], tmpb_v, semb).start()
        pltpu.make_async_copy(shared.at[t, pl.ds(r0, NPT)], tmpa_v, sem2).wait()
        racc_add(tmpa_v)
        pltpu.make_async_copy(shared.at[t + 2, pl.ds(r0, NPT)], tmpa_v, sem2).start()
        pltpu.make_async_copy(shared.at[t + 1, pl.ds(r0, NPT)], tmpb_v, semb).wait()
        racc_add(tmpb_v)
        return 0
    lax.fori_loop(0, (NS - 2) // 2, red_pair, 0)

    pltpu.make_async_copy(shared.at[NS - 1, pl.ds(r0, NPT)], tmpa_v, sem2).wait()
    racc_add(tmpa_v)

    def ep_body(j):
        jo = j * LANES
        racc_v[pl.ds(jo, LANES)] = (racc_v[pl.ds(jo, LANES)]
                                    - 2.0 * gath_v[pl.ds(jo, LANES)]) * SCALE
    plsc.parallel_loop(0, NPT // LANES, unroll=4)(ep_body)

    @pl.when(sid < NS - 1)
    def _():
        pltpu.sync_copy(racc_v, out_hbm.at[pl.ds(r0, NPT)])

    @pl.when(sid == NS - 1)
    def _():
        pltpu.sync_copy(racc_v.at[pl.ds(0, NLAST)],
                        out_hbm.at[pl.ds((NS - 1) * NPT, NLAST)])


@jax.jit
def _sc_scatter(x_flat, ei_blk):
    mesh = plsc.VectorSubcoreMesh(core_axis_name="c", subcore_axis_name="s",
                                  num_cores=1)
    return pl.kernel(
        _sc_body,
        out_type=jax.ShapeDtypeStruct((N_NODES,), jnp.float32),
        mesh=mesh,
        compiler_params=pltpu.CompilerParams(needs_layout_passes=False),
        scratch_types=[
            pltpu.VMEM_SHARED((N_PAD,), jnp.float32),
            pltpu.VMEM((N_PAD,), jnp.float32),
            pltpu.VMEM((NPT // 128, 128), jnp.int32),
            pltpu.VMEM((NPT,), jnp.float32),
            pltpu.VMEM((CPW + 1, 2, 128), jnp.int32),
            pltpu.VMEM((N_PAD,), jnp.float32),
            pltpu.VMEM((NPT,), jnp.float32),
            pltpu.VMEM((NPT,), jnp.float32),
            pltpu.VMEM((NPT,), jnp.float32),
            pltpu.VMEM_SHARED((NS, N_PAD), jnp.float32),
            pltpu.SemaphoreType.DMA,
            pltpu.SemaphoreType.DMA,
            pltpu.SemaphoreType.DMA,
        ],
    )(x_flat, ei_blk)


def kernel(x, edge_index, edge_attr):
    x_flat = x.reshape(-1)
    ei_blk = jnp.transpose(edge_index.reshape(2, ECHUNKS, 128), (1, 0, 2))
    return _sc_scatter(x_flat, ei_blk)
